# baseline (speedup 1.0000x reference)
import functools

import jax
import jax.numpy as jnp
from jax import lax
from jax.experimental import pallas as pl
from jax.experimental.pallas import tpu as pltpu
from jax.experimental.pallas import tpu_sc as plsc

NC = 2
NS = 16
NW = NC * NS
LANES = 16
CHUNK = 256
NPAD = 10240
ROWS_PER_TILE = NPAD // NS
DUMMY = 10000
ZCH = 128

_MESH = dict(core_axis_name="c", subcore_axis_name="s")


def _worker_ids():
    cid = lax.axis_index("c")
    sid = lax.axis_index("s")
    return cid, sid, sid * NC + cid


NSEM_DEG = 4


def _deg_body(dst_hbm, out_hbm, didx_v, ones_v, zb_v, sems, acc_sh):
    cid, sid, wid = _worker_ids()
    nch = dst_hbm.shape[1]
    one = jnp.ones((LANES,), jnp.float32)
    zero = jnp.zeros((LANES,), jnp.float32)
    for i in range(CHUNK // LANES):
        ones_v[pl.ds(i * LANES, LANES)] = one
    for i in range(ROWS_PER_TILE // LANES):
        zb_v[pl.ds(i * LANES, LANES)] = zero
    pltpu.sync_copy(dst_hbm.at[wid], didx_v)
    base = sid * ROWS_PER_TILE
    pltpu.sync_copy(zb_v, acc_sh.at[pl.ds(base, ROWS_PER_TILE)])
    plsc.subcore_barrier()
    pend = [None] * NSEM_DEG
    for j in range(nch):
        b = j % NSEM_DEG
        if pend[b] is not None:
            pend[b].wait()
        pend[b] = pltpu.async_copy(ones_v, acc_sh.at[didx_v.at[j]],
                                   sems.at[b], add=True)
    for b in range(NSEM_DEG):
        if pend[b] is not None:
            pend[b].wait()
    plsc.subcore_barrier()
    pltpu.sync_copy(acc_sh.at[pl.ds(base, ROWS_PER_TILE)], zb_v)
    pltpu.sync_copy(zb_v, out_hbm.at[cid, pl.ds(base, ROWS_PER_TILE)])


def _make_deg(nch):
    @functools.partial(
        pl.kernel,
        out_type=jax.ShapeDtypeStruct((NC, NPAD), jnp.float32),
        mesh=plsc.VectorSubcoreMesh(**_MESH),
        compiler_params=pltpu.CompilerParams(use_tc_tiling_on_sc=False),
        scratch_types=[
            pltpu.VMEM((nch, CHUNK), jnp.int32),
            pltpu.VMEM((CHUNK,), jnp.float32),
            pltpu.VMEM((ROWS_PER_TILE,), jnp.float32),
            pltpu.SemaphoreType.DMA((NSEM_DEG,)),
            pltpu.VMEM_SHARED((NPAD,), jnp.float32),
        ],
    )
    def deg_call(dst_hbm, out_hbm, didx_v, ones_v, zb_v, sems, acc_sh):
        _deg_body(dst_hbm, out_hbm, didx_v, ones_v, zb_v, sems, acc_sh)

    return deg_call


NBUF = 8
DEPTH = 5


def _agg_body(g_hbm, src_hbm, dst_hbm, out_hbm,
              sidx_v, didx_v, rows, zrow_v, wb_v, gsem, ssem, acc_sh):
    cid, sid, wid = _worker_ids()
    nch = src_hbm.shape[1]
    zero = jnp.zeros((LANES,), jnp.float32)
    for i in range(ZCH):
        zrow_v[i, :] = zero
    pltpu.sync_copy(src_hbm.at[wid], sidx_v)
    pltpu.sync_copy(dst_hbm.at[wid], didx_v)
    base = sid * ROWS_PER_TILE
    for k in range(ROWS_PER_TILE // ZCH):
        pltpu.sync_copy(zrow_v, acc_sh.at[pl.ds(base + k * ZCH, ZCH)])
    plsc.subcore_barrier()

    pend_g = [None] * NBUF
    pend_s = [None] * NBUF
    for b in range(min(DEPTH, nch)):
        pend_g[b] = pltpu.async_copy(g_hbm.at[sidx_v.at[b]], rows.at[b],
                                     gsem.at[b])
    for j in range(nch):
        b = j % NBUF
        pend_g[b].wait()
        pend_g[b] = None
        jn = j + DEPTH
        if jn < nch:
            nb = jn % NBUF
            if pend_s[nb] is not None:
                pend_s[nb].wait()
                pend_s[nb] = None
            pend_g[nb] = pltpu.async_copy(g_hbm.at[sidx_v.at[jn]],
                                          rows.at[nb], gsem.at[nb])
        pend_s[b] = pltpu.async_copy(rows.at[b], acc_sh.at[didx_v.at[j]],
                                     ssem.at[b], add=True)
    for b in range(NBUF):
        if pend_s[b] is not None:
            pend_s[b].wait()
    plsc.subcore_barrier()
    pltpu.sync_copy(acc_sh.at[pl.ds(base, ROWS_PER_TILE)], wb_v)
    pltpu.sync_copy(wb_v, out_hbm.at[cid, pl.ds(base, ROWS_PER_TILE)])


def _make_agg(nch):
    @functools.partial(
        pl.kernel,
        out_type=jax.ShapeDtypeStruct((NC, NPAD, LANES), jnp.float32),
        mesh=plsc.VectorSubcoreMesh(**_MESH),
        compiler_params=pltpu.CompilerParams(use_tc_tiling_on_sc=False),
        scratch_types=[
            pltpu.VMEM((nch, CHUNK), jnp.int32),
            pltpu.VMEM((nch, CHUNK), jnp.int32),
            pltpu.VMEM((NBUF, CHUNK, LANES), jnp.float32),
            pltpu.VMEM((ZCH, LANES), jnp.float32),
            pltpu.VMEM((ROWS_PER_TILE, LANES), jnp.float32),
            pltpu.SemaphoreType.DMA((NBUF,)),
            pltpu.SemaphoreType.DMA((NBUF,)),
            pltpu.VMEM_SHARED((NPAD, LANES), jnp.float32),
        ],
    )
    def agg_call(g_hbm, src_hbm, dst_hbm, out_hbm,
                 sidx_v, didx_v, rows, zrow_v, wb_v, gsem, ssem, acc_sh):
        _agg_body(g_hbm, src_hbm, dst_hbm, out_hbm,
                  sidx_v, didx_v, rows, zrow_v, wb_v, gsem, ssem, acc_sh)

    return agg_call


def _tc1_body(degp_ref, x_ref, w1_ref, g1_ref, dis_ref):
    deg = degp_ref[0] + degp_ref[1] + 1.0
    dis = lax.rsqrt(deg)
    h = jnp.dot(x_ref[...], w1_ref[...], preferred_element_type=jnp.float32)
    g1_ref[...] = h * dis
    dis_ref[...] = dis


def _tc2_body(sp_ref, g1_ref, dis_ref, b1_ref, g2_ref):
    s = sp_ref[0] + sp_ref[1]
    out1 = dis_ref[...] * (s + g1_ref[...]) + b1_ref[...]
    g2_ref[...] = dis_ref[...] * jnp.maximum(out1, 0.0)


def _tc3_body(sp_ref, g2_ref, dis_ref, w2_ref, b2_ref, out_ref):
    u = dis_ref[...] * (sp_ref[0] + sp_ref[1] + g2_ref[...])
    o = jnp.dot(u, w2_ref[...], preferred_element_type=jnp.float32) + b2_ref[...]
    m = jnp.max(o, axis=1, keepdims=True)
    lse = m + jnp.log(jnp.sum(jnp.exp(o - m), axis=1, keepdims=True))
    out_ref[...] = o - lse


def _tc1(degp, xp, w1):
    return pl.pallas_call(
        _tc1_body,
        out_shape=[
            jax.ShapeDtypeStruct((NPAD, LANES), jnp.float32),
            jax.ShapeDtypeStruct((NPAD, 1), jnp.float32),
        ],
    )(degp, xp, w1)


def _tc2(sp, g1, dis, b1):
    return pl.pallas_call(
        _tc2_body,
        out_shape=jax.ShapeDtypeStruct((NPAD, LANES), jnp.float32),
    )(sp, g1, dis, b1)


def _tc3(sp, g2, dis, w2p, b2p):
    return pl.pallas_call(
        _tc3_body,
        out_shape=jax.ShapeDtypeStruct((NPAD, LANES), jnp.float32),
    )(sp, g2, dis, w2p, b2p)


def kernel(x, edge_index, W1, b1, W2, b2):
    n, d_in = x.shape
    e = edge_index.shape[1]
    d_out = W2.shape[1]
    nch = -(-e // (NW * CHUNK))
    ep = NW * CHUNK * nch

    ei = edge_index.astype(jnp.int32)
    padv = jnp.full((ep - e,), DUMMY, jnp.int32)
    src3 = jnp.concatenate([ei[0], padv]).reshape(NW, nch, CHUNK)
    dst3 = jnp.concatenate([ei[1], padv]).reshape(NW, nch, CHUNK)
    xp = jnp.pad(x.astype(jnp.float32), ((0, NPAD - n), (0, 0)))

    degp = _make_deg(nch)(dst3).reshape(NC, NPAD, 1)
    g1, dis = _tc1(degp, xp, W1.astype(jnp.float32))

    agg = _make_agg(nch)
    sp1 = agg(g1, src3, dst3)
    g2 = _tc2(sp1, g1, dis, b1.astype(jnp.float32).reshape(1, LANES))
    sp2 = agg(g2, src3, dst3)

    w2p = jnp.pad(W2.astype(jnp.float32), ((0, 0), (0, LANES - d_out)))
    b2p = jnp.concatenate(
        [b2.astype(jnp.float32), jnp.full((LANES - d_out,), -1e30, jnp.float32)]
    ).reshape(1, LANES)
    o = _tc3(sp2, g2, dis, w2p, b2p)
    return o[:n, :d_out]


# speedup vs baseline: 1.2714x; 1.2714x over previous
import functools

import jax
import jax.numpy as jnp
from jax import lax
from jax.experimental import pallas as pl
from jax.experimental.pallas import tpu as pltpu
from jax.experimental.pallas import tpu_sc as plsc

NC = 2
NS = 16
NW = NC * NS
LANES = 16
CHUNK = 128
NPAD = 10240
ROWS_PER_TILE = NPAD // NS
DUMMY = 10000
ZCH = 128

_MESH = dict(core_axis_name="c", subcore_axis_name="s")


def _worker_ids():
    cid = lax.axis_index("c")
    sid = lax.axis_index("s")
    return cid, sid, sid * NC + cid


NSEM_DEG = 4


def _deg_body(dst_hbm, out_hbm, didx_v, ones_v, zb_v, sems, acc_sh):
    cid, sid, wid = _worker_ids()
    nch = dst_hbm.shape[1]
    one = jnp.ones((LANES,), jnp.float32)
    zero = jnp.zeros((LANES,), jnp.float32)
    for i in range(CHUNK // LANES):
        ones_v[pl.ds(i * LANES, LANES)] = one
    for i in range(ROWS_PER_TILE // LANES):
        zb_v[pl.ds(i * LANES, LANES)] = zero
    pltpu.sync_copy(dst_hbm.at[wid], didx_v)
    base = sid * ROWS_PER_TILE
    pltpu.sync_copy(zb_v, acc_sh.at[pl.ds(base, ROWS_PER_TILE)])
    plsc.subcore_barrier()
    pend = [None] * NSEM_DEG
    for j in range(nch):
        b = j % NSEM_DEG
        if pend[b] is not None:
            pend[b].wait()
        pend[b] = pltpu.async_copy(ones_v, acc_sh.at[didx_v.at[j]],
                                   sems.at[b], add=True)
    for b in range(NSEM_DEG):
        if pend[b] is not None:
            pend[b].wait()
    plsc.subcore_barrier()
    pltpu.sync_copy(acc_sh.at[pl.ds(base, ROWS_PER_TILE)], zb_v)
    pltpu.sync_copy(zb_v, out_hbm.at[cid, pl.ds(base, ROWS_PER_TILE)])


def _make_deg(nch):
    @functools.partial(
        pl.kernel,
        out_type=jax.ShapeDtypeStruct((NC, NPAD), jnp.float32),
        mesh=plsc.VectorSubcoreMesh(**_MESH),
        compiler_params=pltpu.CompilerParams(use_tc_tiling_on_sc=False),
        scratch_types=[
            pltpu.VMEM((nch, CHUNK), jnp.int32),
            pltpu.VMEM((CHUNK,), jnp.float32),
            pltpu.VMEM((ROWS_PER_TILE,), jnp.float32),
            pltpu.SemaphoreType.DMA((NSEM_DEG,)),
            pltpu.VMEM_SHARED((NPAD,), jnp.float32),
        ],
    )
    def deg_call(dst_hbm, out_hbm, didx_v, ones_v, zb_v, sems, acc_sh):
        _deg_body(dst_hbm, out_hbm, didx_v, ones_v, zb_v, sems, acc_sh)

    return deg_call


NBUF = 12
DEPTH = 6


def _agg_body(g_hbm, src_hbm, dst_hbm, out_hbm,
              sidx_v, didx_v, rows, zrow_v, wb_v, gsem, ssem, acc_sh):
    cid, sid, wid = _worker_ids()
    nch = src_hbm.shape[1]
    zero = jnp.zeros((LANES,), jnp.float32)
    for i in range(ZCH):
        zrow_v[i, :] = zero
    pltpu.sync_copy(src_hbm.at[wid], sidx_v)
    pltpu.sync_copy(dst_hbm.at[wid], didx_v)
    base = sid * ROWS_PER_TILE
    for k in range(ROWS_PER_TILE // ZCH):
        pltpu.sync_copy(zrow_v, acc_sh.at[pl.ds(base + k * ZCH, ZCH)])
    plsc.subcore_barrier()

    pend_g = [None] * NBUF
    pend_s = [None] * NBUF
    for b in range(min(DEPTH, nch)):
        pend_g[b] = pltpu.async_copy(g_hbm.at[sidx_v.at[b]], rows.at[b],
                                     gsem.at[b])
    for j in range(nch):
        b = j % NBUF
        pend_g[b].wait()
        pend_g[b] = None
        jn = j + DEPTH
        if jn < nch:
            nb = jn % NBUF
            if pend_s[nb] is not None:
                pend_s[nb].wait()
                pend_s[nb] = None
            pend_g[nb] = pltpu.async_copy(g_hbm.at[sidx_v.at[jn]],
                                          rows.at[nb], gsem.at[nb])
        pend_s[b] = pltpu.async_copy(rows.at[b], acc_sh.at[didx_v.at[j]],
                                     ssem.at[b], add=True)
    for b in range(NBUF):
        if pend_s[b] is not None:
            pend_s[b].wait()
    plsc.subcore_barrier()
    pltpu.sync_copy(acc_sh.at[pl.ds(base, ROWS_PER_TILE)], wb_v)
    pltpu.sync_copy(wb_v, out_hbm.at[cid, pl.ds(base, ROWS_PER_TILE)])


def _make_agg(nch):
    @functools.partial(
        pl.kernel,
        out_type=jax.ShapeDtypeStruct((NC, NPAD, LANES), jnp.float32),
        mesh=plsc.VectorSubcoreMesh(**_MESH),
        compiler_params=pltpu.CompilerParams(use_tc_tiling_on_sc=False),
        scratch_types=[
            pltpu.VMEM((nch, CHUNK), jnp.int32),
            pltpu.VMEM((nch, CHUNK), jnp.int32),
            pltpu.VMEM((NBUF, CHUNK, LANES), jnp.float32),
            pltpu.VMEM((ZCH, LANES), jnp.float32),
            pltpu.VMEM((ROWS_PER_TILE, LANES), jnp.float32),
            pltpu.SemaphoreType.DMA((NBUF,)),
            pltpu.SemaphoreType.DMA((NBUF,)),
            pltpu.VMEM_SHARED((NPAD, LANES), jnp.float32),
        ],
    )
    def agg_call(g_hbm, src_hbm, dst_hbm, out_hbm,
                 sidx_v, didx_v, rows, zrow_v, wb_v, gsem, ssem, acc_sh):
        _agg_body(g_hbm, src_hbm, dst_hbm, out_hbm,
                  sidx_v, didx_v, rows, zrow_v, wb_v, gsem, ssem, acc_sh)

    return agg_call


def _tc1_body(degp_ref, x_ref, w1_ref, g1_ref, dis_ref):
    deg = degp_ref[0] + degp_ref[1] + 1.0
    dis = lax.rsqrt(deg)
    h = jnp.dot(x_ref[...], w1_ref[...], preferred_element_type=jnp.float32)
    g1_ref[...] = h * dis
    dis_ref[...] = dis


def _tc2_body(sp_ref, g1_ref, dis_ref, b1_ref, g2_ref):
    s = sp_ref[0] + sp_ref[1]
    out1 = dis_ref[...] * (s + g1_ref[...]) + b1_ref[...]
    g2_ref[...] = dis_ref[...] * jnp.maximum(out1, 0.0)


def _tc3_body(sp_ref, g2_ref, dis_ref, w2_ref, b2_ref, out_ref):
    u = dis_ref[...] * (sp_ref[0] + sp_ref[1] + g2_ref[...])
    o = jnp.dot(u, w2_ref[...], preferred_element_type=jnp.float32) + b2_ref[...]
    m = jnp.max(o, axis=1, keepdims=True)
    lse = m + jnp.log(jnp.sum(jnp.exp(o - m), axis=1, keepdims=True))
    out_ref[...] = o - lse


def _tc1(degp, xp, w1):
    return pl.pallas_call(
        _tc1_body,
        out_shape=[
            jax.ShapeDtypeStruct((NPAD, LANES), jnp.float32),
            jax.ShapeDtypeStruct((NPAD, 1), jnp.float32),
        ],
    )(degp, xp, w1)


def _tc2(sp, g1, dis, b1):
    return pl.pallas_call(
        _tc2_body,
        out_shape=jax.ShapeDtypeStruct((NPAD, LANES), jnp.float32),
    )(sp, g1, dis, b1)


def _tc3(sp, g2, dis, w2p, b2p):
    return pl.pallas_call(
        _tc3_body,
        out_shape=jax.ShapeDtypeStruct((NPAD, LANES), jnp.float32),
    )(sp, g2, dis, w2p, b2p)


def kernel(x, edge_index, W1, b1, W2, b2):
    n, d_in = x.shape
    e = edge_index.shape[1]
    d_out = W2.shape[1]
    nch = -(-e // (NW * CHUNK))
    ep = NW * CHUNK * nch

    ei = edge_index.astype(jnp.int32)
    padv = jnp.full((ep - e,), DUMMY, jnp.int32)
    src3 = jnp.concatenate([ei[0], padv]).reshape(NW, nch, CHUNK)
    dst3 = jnp.concatenate([ei[1], padv]).reshape(NW, nch, CHUNK)
    xp = jnp.pad(x.astype(jnp.float32), ((0, NPAD - n), (0, 0)))

    degp = _make_deg(nch)(dst3).reshape(NC, NPAD, 1)
    g1, dis = _tc1(degp, xp, W1.astype(jnp.float32))

    agg = _make_agg(nch)
    sp1 = agg(g1, src3, dst3)
    g2 = _tc2(sp1, g1, dis, b1.astype(jnp.float32).reshape(1, LANES))
    sp2 = agg(g2, src3, dst3)

    w2p = jnp.pad(W2.astype(jnp.float32), ((0, 0), (0, LANES - d_out)))
    b2p = jnp.concatenate(
        [b2.astype(jnp.float32), jnp.full((LANES - d_out,), -1e30, jnp.float32)]
    ).reshape(1, LANES)
    o = _tc3(sp2, g2, dis, w2p, b2p)
    return o[:n, :d_out]


# speedup vs baseline: 1.2914x; 1.0157x over previous
import functools

import jax
import jax.numpy as jnp
from jax import lax
from jax.experimental import pallas as pl
from jax.experimental.pallas import tpu as pltpu
from jax.experimental.pallas import tpu_sc as plsc

NC = 2
NS = 16
NW = NC * NS
LANES = 16
CHUNK = 128
NPAD = 10240
ROWS_PER_TILE = NPAD // NS
DUMMY = 10000
ZCH = 128

_MESH = dict(core_axis_name="c", subcore_axis_name="s")


def _worker_ids():
    cid = lax.axis_index("c")
    sid = lax.axis_index("s")
    return cid, sid, sid * NC + cid


NSEM_DEG = 8


def _deg_body(dst_hbm, out_hbm, didx_v, ones_v, zb_v, sems, acc_sh):
    cid, sid, wid = _worker_ids()
    nch = dst_hbm.shape[1]
    one = jnp.ones((LANES,), jnp.float32)
    zero = jnp.zeros((LANES,), jnp.float32)
    for i in range(CHUNK // LANES):
        ones_v[pl.ds(i * LANES, LANES)] = one
    for i in range(ROWS_PER_TILE // LANES):
        zb_v[pl.ds(i * LANES, LANES)] = zero
    pltpu.sync_copy(dst_hbm.at[wid], didx_v)
    base = sid * ROWS_PER_TILE
    pltpu.sync_copy(zb_v, acc_sh.at[pl.ds(base, ROWS_PER_TILE)])
    plsc.subcore_barrier()
    pend = [None] * NSEM_DEG
    for j in range(nch):
        b = j % NSEM_DEG
        if pend[b] is not None:
            pend[b].wait()
        pend[b] = pltpu.async_copy(ones_v, acc_sh.at[didx_v.at[j]],
                                   sems.at[b], add=True)
    for b in range(NSEM_DEG):
        if pend[b] is not None:
            pend[b].wait()
    plsc.subcore_barrier()
    pltpu.sync_copy(acc_sh.at[pl.ds(base, ROWS_PER_TILE)], zb_v)
    pltpu.sync_copy(zb_v, out_hbm.at[cid, pl.ds(base, ROWS_PER_TILE)])


def _make_deg(nch):
    @functools.partial(
        pl.kernel,
        out_type=jax.ShapeDtypeStruct((NC, NPAD), jnp.float32),
        mesh=plsc.VectorSubcoreMesh(**_MESH),
        compiler_params=pltpu.CompilerParams(use_tc_tiling_on_sc=False),
        scratch_types=[
            pltpu.VMEM((nch, CHUNK), jnp.int32),
            pltpu.VMEM((CHUNK,), jnp.float32),
            pltpu.VMEM((ROWS_PER_TILE,), jnp.float32),
            pltpu.SemaphoreType.DMA((NSEM_DEG,)),
            pltpu.VMEM_SHARED((NPAD,), jnp.float32),
        ],
    )
    def deg_call(dst_hbm, out_hbm, didx_v, ones_v, zb_v, sems, acc_sh):
        _deg_body(dst_hbm, out_hbm, didx_v, ones_v, zb_v, sems, acc_sh)

    return deg_call


NBUF = 12
DEPTH = 6


def _agg_body(g_hbm, src_hbm, dst_hbm, out_hbm,
              sidx_v, didx_v, rows, zrow_v, wb_v, gsem, ssem, acc_sh):
    cid, sid, wid = _worker_ids()
    nch = src_hbm.shape[1]
    zero = jnp.zeros((LANES,), jnp.float32)
    for i in range(ZCH):
        zrow_v[i, :] = zero
    pltpu.sync_copy(src_hbm.at[wid], sidx_v)
    pltpu.sync_copy(dst_hbm.at[wid], didx_v)
    base = sid * ROWS_PER_TILE
    for k in range(ROWS_PER_TILE // ZCH):
        pltpu.sync_copy(zrow_v, acc_sh.at[pl.ds(base + k * ZCH, ZCH)])
    plsc.subcore_barrier()

    pend_g = [None] * NBUF
    pend_s = [None] * NBUF
    for b in range(min(DEPTH, nch)):
        pend_g[b] = pltpu.async_copy(g_hbm.at[sidx_v.at[b]], rows.at[b],
                                     gsem.at[b])
    for j in range(nch):
        b = j % NBUF
        pend_g[b].wait()
        pend_g[b] = None
        jn = j + DEPTH
        if jn < nch:
            nb = jn % NBUF
            if pend_s[nb] is not None:
                pend_s[nb].wait()
                pend_s[nb] = None
            pend_g[nb] = pltpu.async_copy(g_hbm.at[sidx_v.at[jn]],
                                          rows.at[nb], gsem.at[nb])
        pend_s[b] = pltpu.async_copy(rows.at[b], acc_sh.at[didx_v.at[j]],
                                     ssem.at[b], add=True)
    for b in range(NBUF):
        if pend_s[b] is not None:
            pend_s[b].wait()
    plsc.subcore_barrier()
    pltpu.sync_copy(acc_sh.at[pl.ds(base, ROWS_PER_TILE)], wb_v)
    pltpu.sync_copy(wb_v, out_hbm.at[cid, pl.ds(base, ROWS_PER_TILE)])


def _make_agg(nch):
    @functools.partial(
        pl.kernel,
        out_type=jax.ShapeDtypeStruct((NC, NPAD, LANES), jnp.float32),
        mesh=plsc.VectorSubcoreMesh(**_MESH),
        compiler_params=pltpu.CompilerParams(use_tc_tiling_on_sc=False),
        scratch_types=[
            pltpu.VMEM((nch, CHUNK), jnp.int32),
            pltpu.VMEM((nch, CHUNK), jnp.int32),
            pltpu.VMEM((NBUF, CHUNK, LANES), jnp.float32),
            pltpu.VMEM((ZCH, LANES), jnp.float32),
            pltpu.VMEM((ROWS_PER_TILE, LANES), jnp.float32),
            pltpu.SemaphoreType.DMA((NBUF,)),
            pltpu.SemaphoreType.DMA((NBUF,)),
            pltpu.VMEM_SHARED((NPAD, LANES), jnp.float32),
        ],
    )
    def agg_call(g_hbm, src_hbm, dst_hbm, out_hbm,
                 sidx_v, didx_v, rows, zrow_v, wb_v, gsem, ssem, acc_sh):
        _agg_body(g_hbm, src_hbm, dst_hbm, out_hbm,
                  sidx_v, didx_v, rows, zrow_v, wb_v, gsem, ssem, acc_sh)

    return agg_call


def _tcmm_body(x_ref, w1_ref, h_ref):
    h_ref[...] = jnp.dot(x_ref[...], w1_ref[...],
                         preferred_element_type=jnp.float32)


def _tc1_body(degp_ref, h_ref, g1_ref, dis_ref):
    deg = degp_ref[0] + degp_ref[1] + 1.0
    dis = lax.rsqrt(deg)
    g1_ref[...] = h_ref[...] * dis
    dis_ref[...] = dis


def _tc2_body(sp_ref, g1_ref, dis_ref, b1_ref, g2_ref):
    s = sp_ref[0] + sp_ref[1]
    out1 = dis_ref[...] * (s + g1_ref[...]) + b1_ref[...]
    g2_ref[...] = dis_ref[...] * jnp.maximum(out1, 0.0)


def _tc3_body(sp_ref, g2_ref, dis_ref, w2_ref, b2_ref, out_ref):
    u = dis_ref[...] * (sp_ref[0] + sp_ref[1] + g2_ref[...])
    o = jnp.dot(u, w2_ref[...], preferred_element_type=jnp.float32) + b2_ref[...]
    m = jnp.max(o, axis=1, keepdims=True)
    lse = m + jnp.log(jnp.sum(jnp.exp(o - m), axis=1, keepdims=True))
    out_ref[...] = o - lse


def _tcmm(xp, w1):
    return pl.pallas_call(
        _tcmm_body,
        out_shape=jax.ShapeDtypeStruct((NPAD, LANES), jnp.float32),
    )(xp, w1)


def _tc1(degp, h):
    return pl.pallas_call(
        _tc1_body,
        out_shape=[
            jax.ShapeDtypeStruct((NPAD, LANES), jnp.float32),
            jax.ShapeDtypeStruct((NPAD, 1), jnp.float32),
        ],
    )(degp, h)


def _tc2(sp, g1, dis, b1):
    return pl.pallas_call(
        _tc2_body,
        out_shape=jax.ShapeDtypeStruct((NPAD, LANES), jnp.float32),
    )(sp, g1, dis, b1)


def _tc3(sp, g2, dis, w2p, b2p):
    return pl.pallas_call(
        _tc3_body,
        out_shape=jax.ShapeDtypeStruct((NPAD, LANES), jnp.float32),
    )(sp, g2, dis, w2p, b2p)


def kernel(x, edge_index, W1, b1, W2, b2):
    n, d_in = x.shape
    e = edge_index.shape[1]
    d_out = W2.shape[1]
    nch = -(-e // (NW * CHUNK))
    ep = NW * CHUNK * nch

    ei = edge_index.astype(jnp.int32)
    padv = jnp.full((ep - e,), DUMMY, jnp.int32)
    src3 = jnp.concatenate([ei[0], padv]).reshape(NW, nch, CHUNK)
    dst3 = jnp.concatenate([ei[1], padv]).reshape(NW, nch, CHUNK)
    xp = jnp.pad(x.astype(jnp.float32), ((0, NPAD - n), (0, 0)))

    h1 = _tcmm(xp, W1.astype(jnp.float32))
    degp = _make_deg(nch)(dst3).reshape(NC, NPAD, 1)
    g1, dis = _tc1(degp, h1)

    agg = _make_agg(nch)
    sp1 = agg(g1, src3, dst3)
    g2 = _tc2(sp1, g1, dis, b1.astype(jnp.float32).reshape(1, LANES))
    sp2 = agg(g2, src3, dst3)

    w2p = jnp.pad(W2.astype(jnp.float32), ((0, 0), (0, LANES - d_out)))
    b2p = jnp.concatenate(
        [b2.astype(jnp.float32), jnp.full((LANES - d_out,), -1e30, jnp.float32)]
    ).reshape(1, LANES)
    o = _tc3(sp2, g2, dis, w2p, b2p)
    return o[:n, :d_out]


# speedup vs baseline: 1.3058x; 1.0111x over previous
import functools

import jax
import jax.numpy as jnp
from jax import lax
from jax.experimental import pallas as pl
from jax.experimental.pallas import tpu as pltpu
from jax.experimental.pallas import tpu_sc as plsc

NC = 2
NS = 16
NW = NC * NS
LANES = 16
CHUNK = 128
NPAD = 10240
ROWS_PER_TILE = NPAD // NS
DUMMY = 10000
ZCH = 128

_MESH = dict(core_axis_name="c", subcore_axis_name="s")


def _worker_ids():
    cid = lax.axis_index("c")
    sid = lax.axis_index("s")
    return cid, sid, sid * NC + cid


NSEM_DEG = 8


def _deg_body(dst_hbm, out_hbm, didx_v, ones_v, zb_v, sems, acc_sh):
    cid, sid, wid = _worker_ids()
    nch = dst_hbm.shape[1]
    one = jnp.ones((LANES,), jnp.float32)
    zero = jnp.zeros((LANES,), jnp.float32)
    cp_di = pltpu.async_copy(dst_hbm.at[wid], didx_v, sems.at[NSEM_DEG])
    for i in range(CHUNK // LANES):
        ones_v[pl.ds(i * LANES, LANES)] = one
    for i in range(ROWS_PER_TILE // LANES):
        zb_v[pl.ds(i * LANES, LANES)] = zero
    base = sid * ROWS_PER_TILE
    pltpu.sync_copy(zb_v, acc_sh.at[pl.ds(base, ROWS_PER_TILE)])
    cp_di.wait()
    plsc.subcore_barrier()
    pend = [None] * NSEM_DEG
    for j in range(nch):
        b = j % NSEM_DEG
        if pend[b] is not None:
            pend[b].wait()
        pend[b] = pltpu.async_copy(ones_v, acc_sh.at[didx_v.at[j]],
                                   sems.at[b], add=True)
    for b in range(NSEM_DEG):
        if pend[b] is not None:
            pend[b].wait()
    plsc.subcore_barrier()
    pltpu.sync_copy(acc_sh.at[pl.ds(base, ROWS_PER_TILE)], zb_v)
    pltpu.sync_copy(zb_v, out_hbm.at[cid, pl.ds(base, ROWS_PER_TILE)])


def _make_deg(nch):
    @functools.partial(
        pl.kernel,
        out_type=jax.ShapeDtypeStruct((NC, NPAD), jnp.float32),
        mesh=plsc.VectorSubcoreMesh(**_MESH),
        compiler_params=pltpu.CompilerParams(use_tc_tiling_on_sc=False),
        scratch_types=[
            pltpu.VMEM((nch, CHUNK), jnp.int32),
            pltpu.VMEM((CHUNK,), jnp.float32),
            pltpu.VMEM((ROWS_PER_TILE,), jnp.float32),
            pltpu.SemaphoreType.DMA((NSEM_DEG + 1,)),
            pltpu.VMEM_SHARED((NPAD,), jnp.float32),
        ],
    )
    def deg_call(dst_hbm, out_hbm, didx_v, ones_v, zb_v, sems, acc_sh):
        _deg_body(dst_hbm, out_hbm, didx_v, ones_v, zb_v, sems, acc_sh)

    return deg_call


NBUF = 12
DEPTH = 6


def _agg_body(g_hbm, src_hbm, dst_hbm, out_hbm,
              sidx_v, didx_v, rows, zrow_v, wb_v, gsem, ssem, acc_sh):
    cid, sid, wid = _worker_ids()
    nch = src_hbm.shape[1]
    cp_si = pltpu.async_copy(src_hbm.at[wid], sidx_v, gsem.at[NBUF])
    cp_di = pltpu.async_copy(dst_hbm.at[wid], didx_v, gsem.at[NBUF + 1])
    zero = jnp.zeros((LANES,), jnp.float32)
    for i in range(ZCH):
        zrow_v[i, :] = zero
    base = sid * ROWS_PER_TILE
    for k in range(ROWS_PER_TILE // ZCH):
        pltpu.sync_copy(zrow_v, acc_sh.at[pl.ds(base + k * ZCH, ZCH)])
    cp_si.wait()
    cp_di.wait()
    plsc.subcore_barrier()

    pend_g = [None] * NBUF
    pend_s = [None] * NBUF
    for b in range(min(DEPTH, nch)):
        pend_g[b] = pltpu.async_copy(g_hbm.at[sidx_v.at[b]], rows.at[b],
                                     gsem.at[b])
    for j in range(nch):
        b = j % NBUF
        pend_g[b].wait()
        pend_g[b] = None
        jn = j + DEPTH
        if jn < nch:
            nb = jn % NBUF
            if pend_s[nb] is not None:
                pend_s[nb].wait()
                pend_s[nb] = None
            pend_g[nb] = pltpu.async_copy(g_hbm.at[sidx_v.at[jn]],
                                          rows.at[nb], gsem.at[nb])
        pend_s[b] = pltpu.async_copy(rows.at[b], acc_sh.at[didx_v.at[j]],
                                     ssem.at[b], add=True)
    for b in range(NBUF):
        if pend_s[b] is not None:
            pend_s[b].wait()
    plsc.subcore_barrier()
    pltpu.sync_copy(acc_sh.at[pl.ds(base, ROWS_PER_TILE)], wb_v)
    pltpu.sync_copy(wb_v, out_hbm.at[cid, pl.ds(base, ROWS_PER_TILE)])


def _make_agg(nch):
    @functools.partial(
        pl.kernel,
        out_type=jax.ShapeDtypeStruct((NC, NPAD, LANES), jnp.float32),
        mesh=plsc.VectorSubcoreMesh(**_MESH),
        compiler_params=pltpu.CompilerParams(use_tc_tiling_on_sc=False),
        scratch_types=[
            pltpu.VMEM((nch, CHUNK), jnp.int32),
            pltpu.VMEM((nch, CHUNK), jnp.int32),
            pltpu.VMEM((NBUF, CHUNK, LANES), jnp.float32),
            pltpu.VMEM((ZCH, LANES), jnp.float32),
            pltpu.VMEM((ROWS_PER_TILE, LANES), jnp.float32),
            pltpu.SemaphoreType.DMA((NBUF + 2,)),
            pltpu.SemaphoreType.DMA((NBUF,)),
            pltpu.VMEM_SHARED((NPAD, LANES), jnp.float32),
        ],
    )
    def agg_call(g_hbm, src_hbm, dst_hbm, out_hbm,
                 sidx_v, didx_v, rows, zrow_v, wb_v, gsem, ssem, acc_sh):
        _agg_body(g_hbm, src_hbm, dst_hbm, out_hbm,
                  sidx_v, didx_v, rows, zrow_v, wb_v, gsem, ssem, acc_sh)

    return agg_call


def _tcmm_body(x_ref, w1_ref, h_ref):
    h_ref[...] = jnp.dot(x_ref[...], w1_ref[...],
                         preferred_element_type=jnp.float32)


def _tc1_body(degp_ref, h_ref, g1_ref, dis_ref):
    deg = degp_ref[0] + degp_ref[1] + 1.0
    dis = lax.rsqrt(deg)
    g1_ref[...] = h_ref[...] * dis
    dis_ref[...] = dis


def _tc2_body(sp_ref, g1_ref, dis_ref, b1_ref, g2_ref):
    s = sp_ref[0] + sp_ref[1]
    out1 = dis_ref[...] * (s + g1_ref[...]) + b1_ref[...]
    g2_ref[...] = dis_ref[...] * jnp.maximum(out1, 0.0)


def _tc3_body(sp_ref, g2_ref, dis_ref, w2_ref, b2_ref, out_ref):
    u = dis_ref[...] * (sp_ref[0] + sp_ref[1] + g2_ref[...])
    o = jnp.dot(u, w2_ref[...], preferred_element_type=jnp.float32) + b2_ref[...]
    m = jnp.max(o, axis=1, keepdims=True)
    lse = m + jnp.log(jnp.sum(jnp.exp(o - m), axis=1, keepdims=True))
    out_ref[...] = o - lse


def _tcmm(xp, w1):
    return pl.pallas_call(
        _tcmm_body,
        out_shape=jax.ShapeDtypeStruct((NPAD, LANES), jnp.float32),
    )(xp, w1)


def _tc1(degp, h):
    return pl.pallas_call(
        _tc1_body,
        out_shape=[
            jax.ShapeDtypeStruct((NPAD, LANES), jnp.float32),
            jax.ShapeDtypeStruct((NPAD, 1), jnp.float32),
        ],
    )(degp, h)


def _tc2(sp, g1, dis, b1):
    return pl.pallas_call(
        _tc2_body,
        out_shape=jax.ShapeDtypeStruct((NPAD, LANES), jnp.float32),
    )(sp, g1, dis, b1)


def _tc3(sp, g2, dis, w2p, b2p):
    return pl.pallas_call(
        _tc3_body,
        out_shape=jax.ShapeDtypeStruct((NPAD, LANES), jnp.float32),
    )(sp, g2, dis, w2p, b2p)


def kernel(x, edge_index, W1, b1, W2, b2):
    n, d_in = x.shape
    e = edge_index.shape[1]
    d_out = W2.shape[1]
    nch = -(-e // (NW * CHUNK))
    ep = NW * CHUNK * nch

    ei = edge_index.astype(jnp.int32)
    padv = jnp.full((ep - e,), DUMMY, jnp.int32)
    src3 = jnp.concatenate([ei[0], padv]).reshape(NW, nch, CHUNK)
    dst3 = jnp.concatenate([ei[1], padv]).reshape(NW, nch, CHUNK)
    xp = jnp.pad(x.astype(jnp.float32), ((0, NPAD - n), (0, 0)))

    h1 = _tcmm(xp, W1.astype(jnp.float32))
    degp = _make_deg(nch)(dst3).reshape(NC, NPAD, 1)
    g1, dis = _tc1(degp, h1)

    agg = _make_agg(nch)
    sp1 = agg(g1, src3, dst3)
    g2 = _tc2(sp1, g1, dis, b1.astype(jnp.float32).reshape(1, LANES))
    sp2 = agg(g2, src3, dst3)

    w2p = jnp.pad(W2.astype(jnp.float32), ((0, 0), (0, LANES - d_out)))
    b2p = jnp.concatenate(
        [b2.astype(jnp.float32), jnp.full((LANES - d_out,), -1e30, jnp.float32)]
    ).reshape(1, LANES)
    o = _tc3(sp2, g2, dis, w2p, b2p)
    return o[:n, :d_out]


# speedup vs baseline: 1.5997x; 1.2251x over previous
import functools

import jax
import jax.numpy as jnp
from jax import lax
from jax.experimental import pallas as pl
from jax.experimental.pallas import tpu as pltpu
from jax.experimental.pallas import tpu_sc as plsc

NC = 2
NS = 16
NW = NC * NS
LANES = 16
CHUNK = 128
NPAD = 10240
ROWS_PER_TILE = NPAD // NS
DUMMY = 10000
ZCH = 128

_MESH = dict(core_axis_name="c", subcore_axis_name="s")


def _worker_ids():
    cid = lax.axis_index("c")
    sid = lax.axis_index("s")
    return cid, sid, sid * NC + cid


NSEM_DEG = 8


def _deg_body(dst_hbm, out_hbm, didx_v, ones_v, zb_v, sems, acc_sh):
    cid, sid, wid = _worker_ids()
    nch = dst_hbm.shape[1]
    one = jnp.ones((LANES,), jnp.float32)
    zero = jnp.zeros((LANES,), jnp.float32)
    cp_di = pltpu.async_copy(dst_hbm.at[wid], didx_v, sems.at[NSEM_DEG])
    for i in range(CHUNK // LANES):
        ones_v[pl.ds(i * LANES, LANES)] = one
    for i in range(ROWS_PER_TILE // LANES):
        zb_v[pl.ds(i * LANES, LANES)] = zero
    base = sid * ROWS_PER_TILE
    pltpu.sync_copy(zb_v, acc_sh.at[pl.ds(base, ROWS_PER_TILE)])
    cp_di.wait()
    plsc.subcore_barrier()
    pend = [None] * NSEM_DEG
    for j in range(nch):
        b = j % NSEM_DEG
        if pend[b] is not None:
            pend[b].wait()
        pend[b] = pltpu.async_copy(ones_v, acc_sh.at[didx_v.at[j]],
                                   sems.at[b], add=True)
    for b in range(NSEM_DEG):
        if pend[b] is not None:
            pend[b].wait()
    plsc.subcore_barrier()
    pltpu.sync_copy(acc_sh.at[pl.ds(base, ROWS_PER_TILE)], zb_v)
    pltpu.sync_copy(zb_v, out_hbm.at[cid, pl.ds(base, ROWS_PER_TILE)])


def _make_deg(nch):
    @functools.partial(
        pl.kernel,
        out_type=jax.ShapeDtypeStruct((NC, NPAD), jnp.float32),
        mesh=plsc.VectorSubcoreMesh(**_MESH),
        compiler_params=pltpu.CompilerParams(use_tc_tiling_on_sc=False),
        scratch_types=[
            pltpu.VMEM((nch, CHUNK), jnp.int32),
            pltpu.VMEM((CHUNK,), jnp.float32),
            pltpu.VMEM((ROWS_PER_TILE,), jnp.float32),
            pltpu.SemaphoreType.DMA((NSEM_DEG + 1,)),
            pltpu.VMEM_SHARED((NPAD,), jnp.float32),
        ],
    )
    def deg_call(dst_hbm, out_hbm, didx_v, ones_v, zb_v, sems, acc_sh):
        _deg_body(dst_hbm, out_hbm, didx_v, ones_v, zb_v, sems, acc_sh)

    return deg_call


NBUF = 12
DEPTH = 6


def _agg_body(g_hbm, src_hbm, dst_hbm, out_hbm,
              sidx_v, didx_v, rows, zrow_v, wb_v, gsem, ssem, acc_sh, g_sh):
    cid, sid, wid = _worker_ids()
    nch = src_hbm.shape[1]
    cp_si = pltpu.async_copy(src_hbm.at[wid], sidx_v, gsem.at[NBUF])
    cp_di = pltpu.async_copy(dst_hbm.at[wid], didx_v, gsem.at[NBUF + 1])
    zero = jnp.zeros((LANES,), jnp.float32)
    for i in range(ZCH):
        zrow_v[i, :] = zero
    base = sid * ROWS_PER_TILE
    pltpu.sync_copy(g_hbm.at[pl.ds(base, ROWS_PER_TILE)], wb_v)
    pltpu.sync_copy(wb_v, g_sh.at[pl.ds(base, ROWS_PER_TILE)])
    for k in range(ROWS_PER_TILE // ZCH):
        pltpu.sync_copy(zrow_v, acc_sh.at[pl.ds(base + k * ZCH, ZCH)])
    cp_si.wait()
    cp_di.wait()
    plsc.subcore_barrier()

    pend_g = [None] * NBUF
    pend_s = [None] * NBUF
    for b in range(min(DEPTH, nch)):
        pend_g[b] = pltpu.async_copy(g_sh.at[sidx_v.at[b]], rows.at[b],
                                     gsem.at[b])
    for j in range(nch):
        b = j % NBUF
        pend_g[b].wait()
        pend_g[b] = None
        jn = j + DEPTH
        if jn < nch:
            nb = jn % NBUF
            if pend_s[nb] is not None:
                pend_s[nb].wait()
                pend_s[nb] = None
            pend_g[nb] = pltpu.async_copy(g_sh.at[sidx_v.at[jn]],
                                          rows.at[nb], gsem.at[nb])
        pend_s[b] = pltpu.async_copy(rows.at[b], acc_sh.at[didx_v.at[j]],
                                     ssem.at[b], add=True)
    for b in range(NBUF):
        if pend_s[b] is not None:
            pend_s[b].wait()
    plsc.subcore_barrier()
    pltpu.sync_copy(acc_sh.at[pl.ds(base, ROWS_PER_TILE)], wb_v)
    pltpu.sync_copy(wb_v, out_hbm.at[cid, pl.ds(base, ROWS_PER_TILE)])


def _make_agg(nch):
    @functools.partial(
        pl.kernel,
        out_type=jax.ShapeDtypeStruct((NC, NPAD, LANES), jnp.float32),
        mesh=plsc.VectorSubcoreMesh(**_MESH),
        compiler_params=pltpu.CompilerParams(use_tc_tiling_on_sc=False),
        scratch_types=[
            pltpu.VMEM((nch, CHUNK), jnp.int32),
            pltpu.VMEM((nch, CHUNK), jnp.int32),
            pltpu.VMEM((NBUF, CHUNK, LANES), jnp.float32),
            pltpu.VMEM((ZCH, LANES), jnp.float32),
            pltpu.VMEM((ROWS_PER_TILE, LANES), jnp.float32),
            pltpu.SemaphoreType.DMA((NBUF + 2,)),
            pltpu.SemaphoreType.DMA((NBUF,)),
            pltpu.VMEM_SHARED((NPAD, LANES), jnp.float32),
            pltpu.VMEM_SHARED((NPAD, LANES), jnp.float32),
        ],
    )
    def agg_call(g_hbm, src_hbm, dst_hbm, out_hbm,
                 sidx_v, didx_v, rows, zrow_v, wb_v, gsem, ssem, acc_sh, g_sh):
        _agg_body(g_hbm, src_hbm, dst_hbm, out_hbm,
                  sidx_v, didx_v, rows, zrow_v, wb_v, gsem, ssem, acc_sh, g_sh)

    return agg_call


def _tcmm_body(x_ref, w1_ref, h_ref):
    h_ref[...] = jnp.dot(x_ref[...], w1_ref[...],
                         preferred_element_type=jnp.float32)


def _tc1_body(degp_ref, h_ref, g1_ref, dis_ref):
    deg = degp_ref[0] + degp_ref[1] + 1.0
    dis = lax.rsqrt(deg)
    g1_ref[...] = h_ref[...] * dis
    dis_ref[...] = dis


def _tc2_body(sp_ref, g1_ref, dis_ref, b1_ref, g2_ref):
    s = sp_ref[0] + sp_ref[1]
    out1 = dis_ref[...] * (s + g1_ref[...]) + b1_ref[...]
    g2_ref[...] = dis_ref[...] * jnp.maximum(out1, 0.0)


def _tc3_body(sp_ref, g2_ref, dis_ref, w2_ref, b2_ref, out_ref):
    u = dis_ref[...] * (sp_ref[0] + sp_ref[1] + g2_ref[...])
    o = jnp.dot(u, w2_ref[...], preferred_element_type=jnp.float32) + b2_ref[...]
    m = jnp.max(o, axis=1, keepdims=True)
    lse = m + jnp.log(jnp.sum(jnp.exp(o - m), axis=1, keepdims=True))
    out_ref[...] = o - lse


def _tcmm(xp, w1):
    return pl.pallas_call(
        _tcmm_body,
        out_shape=jax.ShapeDtypeStruct((NPAD, LANES), jnp.float32),
    )(xp, w1)


def _tc1(degp, h):
    return pl.pallas_call(
        _tc1_body,
        out_shape=[
            jax.ShapeDtypeStruct((NPAD, LANES), jnp.float32),
            jax.ShapeDtypeStruct((NPAD, 1), jnp.float32),
        ],
    )(degp, h)


def _tc2(sp, g1, dis, b1):
    return pl.pallas_call(
        _tc2_body,
        out_shape=jax.ShapeDtypeStruct((NPAD, LANES), jnp.float32),
    )(sp, g1, dis, b1)


def _tc3(sp, g2, dis, w2p, b2p):
    return pl.pallas_call(
        _tc3_body,
        out_shape=jax.ShapeDtypeStruct((NPAD, LANES), jnp.float32),
    )(sp, g2, dis, w2p, b2p)


def kernel(x, edge_index, W1, b1, W2, b2):
    n, d_in = x.shape
    e = edge_index.shape[1]
    d_out = W2.shape[1]
    nch = -(-e // (NW * CHUNK))
    ep = NW * CHUNK * nch

    ei = edge_index.astype(jnp.int32)
    padv = jnp.full((ep - e,), DUMMY, jnp.int32)
    src3 = jnp.concatenate([ei[0], padv]).reshape(NW, nch, CHUNK)
    dst3 = jnp.concatenate([ei[1], padv]).reshape(NW, nch, CHUNK)
    xp = jnp.pad(x.astype(jnp.float32), ((0, NPAD - n), (0, 0)))

    h1 = _tcmm(xp, W1.astype(jnp.float32))
    degp = _make_deg(nch)(dst3).reshape(NC, NPAD, 1)
    g1, dis = _tc1(degp, h1)

    agg = _make_agg(nch)
    sp1 = agg(g1, src3, dst3)
    g2 = _tc2(sp1, g1, dis, b1.astype(jnp.float32).reshape(1, LANES))
    sp2 = agg(g2, src3, dst3)

    w2p = jnp.pad(W2.astype(jnp.float32), ((0, 0), (0, LANES - d_out)))
    b2p = jnp.concatenate(
        [b2.astype(jnp.float32), jnp.full((LANES - d_out,), -1e30, jnp.float32)]
    ).reshape(1, LANES)
    o = _tc3(sp2, g2, dis, w2p, b2p)
    return o[:n, :d_out]
